# baseline (speedup 1.0000x reference)
import functools

import jax
import jax.numpy as jnp
from jax import lax
from jax.experimental import pallas as pl
from jax.experimental.pallas import tpu as pltpu
from jax.experimental.pallas import tpu_sc as plsc

NC = 2
NS = 16
NW = NC * NS

WORD_DIM = 100
WORD_PAD = 104
LABEL_DIM = 32
PACK_DIM = WORD_PAD + LABEL_DIM

CHUNK = 128


@functools.partial(jax.jit, static_argnames=("n_rows",))
def _embed_concat(word_table_pad, label_table, word_ids, label_ids, *, n_rows):
    rows_per_w = n_rows // NW
    n_chunks = rows_per_w // CHUNK
    mesh = plsc.VectorSubcoreMesh(
        core_axis_name="c", subcore_axis_name="s", num_cores=NC, num_subcores=NS
    )

    @functools.partial(
        pl.kernel,
        out_type=jax.ShapeDtypeStruct((n_rows, PACK_DIM), jnp.float32),
        mesh=mesh,
        compiler_params=pltpu.CompilerParams(
            use_tc_tiling_on_sc=False, needs_layout_passes=False
        ),
        scratch_types=[
            pltpu.VMEM((CHUNK,), jnp.int32),
            pltpu.VMEM((CHUNK,), jnp.int32),
            pltpu.VMEM((CHUNK, WORD_PAD), jnp.float32),
            pltpu.VMEM((CHUNK, LABEL_DIM), jnp.float32),
            pltpu.SemaphoreType.DMA,
            pltpu.SemaphoreType.DMA,
        ],
    )
    def body(wtab, ltab, wids, lids, out, widx, lidx, wrows, lrows, sem0, sem1):
        wid = lax.axis_index("s") * NC + lax.axis_index("c")
        base0 = wid * rows_per_w

        def step(i, carry):
            base = base0 + i * CHUNK
            pltpu.sync_copy(wids.at[pl.ds(base, CHUNK)], widx)
            pltpu.sync_copy(lids.at[pl.ds(base, CHUNK)], lidx)
            cw = pltpu.async_copy(wtab.at[widx], wrows, sem0)
            cl = pltpu.async_copy(ltab.at[lidx], lrows, sem1)
            cw.wait()
            cl.wait()
            pltpu.sync_copy(wrows, out.at[pl.ds(base, CHUNK), pl.ds(0, WORD_PAD)])
            pltpu.sync_copy(lrows, out.at[pl.ds(base, CHUNK), pl.ds(WORD_PAD, LABEL_DIM)])
            return carry

        lax.fori_loop(0, n_chunks, step, 0)

    return body(word_table_pad, label_table, word_ids, label_ids)


_PAD_BLOCK = 4096


def _pad_body(x_ref, o_ref):
    o_ref[...] = jnp.concatenate(
        [x_ref[...], jnp.zeros((x_ref.shape[0], WORD_PAD - WORD_DIM), jnp.float32)],
        axis=1,
    )


def _pad_tc(word_table):
    v = word_table.shape[0]
    grid = (v + _PAD_BLOCK - 1) // _PAD_BLOCK
    return pl.pallas_call(
        _pad_body,
        grid=(grid,),
        in_specs=[pl.BlockSpec((_PAD_BLOCK, WORD_DIM), lambda i: (i, 0))],
        out_specs=pl.BlockSpec((_PAD_BLOCK, WORD_PAD), lambda i: (i, 0)),
        out_shape=jax.ShapeDtypeStruct((v, WORD_PAD), jnp.float32),
    )(word_table)


_STRIP_BLOCK = 2048


def _strip_body(x_ref, o_ref):
    x = x_ref[...]
    o_ref[...] = jnp.concatenate([x[:, :WORD_DIM], x[:, WORD_PAD:]], axis=1)


def _strip_tc(packed):
    n = packed.shape[0]
    grid = n // _STRIP_BLOCK
    return pl.pallas_call(
        _strip_body,
        grid=(grid,),
        in_specs=[pl.BlockSpec((_STRIP_BLOCK, PACK_DIM), lambda i: (i, 0))],
        out_specs=pl.BlockSpec((_STRIP_BLOCK, WORD_DIM + LABEL_DIM), lambda i: (i, 0)),
        out_shape=jax.ShapeDtypeStruct((n, WORD_DIM + LABEL_DIM), jnp.float32),
    )(packed)


_EXT_ROWS = 128


def _extract_body(x_ref, w_ref, l_ref):
    x = x_ref[...]
    w_ref[...] = x[:, :, 0].reshape(w_ref.shape)
    l_ref[...] = x[:, :, 1].reshape(l_ref.shape)


def _extract_ids_tc(inputs):
    b, l, _ = inputs.shape
    grid = b // _EXT_ROWS
    blk = _EXT_ROWS * l
    out_shapes = (
        jax.ShapeDtypeStruct((b * l,), jnp.int32),
        jax.ShapeDtypeStruct((b * l,), jnp.int32),
    )
    return pl.pallas_call(
        _extract_body,
        grid=(grid,),
        in_specs=[pl.BlockSpec((_EXT_ROWS, l, 2), lambda i: (i, 0, 0))],
        out_specs=(
            pl.BlockSpec((blk,), lambda i: (i,)),
            pl.BlockSpec((blk,), lambda i: (i,)),
        ),
        out_shape=out_shapes,
    )(inputs)


def kernel(inputs, word_table, label_table):
    b, l, _ = inputs.shape
    word_ids, label_ids = _extract_ids_tc(inputs)
    wt_pad = _pad_tc(word_table)
    packed = _embed_concat(wt_pad, label_table, word_ids, label_ids, n_rows=b * l)
    out = _strip_tc(packed)
    return out.reshape(b, l, WORD_DIM + LABEL_DIM)


# speedup vs baseline: 1.1069x; 1.1069x over previous
import functools

import jax
import jax.numpy as jnp
from jax import lax
from jax.experimental import pallas as pl
from jax.experimental.pallas import tpu as pltpu
from jax.experimental.pallas import tpu_sc as plsc

NC = 2
NS = 16
NW = NC * NS

WORD_DIM = 100
WORD_PAD = 104
LABEL_DIM = 32
PACK_DIM = WORD_PAD + LABEL_DIM

CHUNK = 128


@functools.partial(jax.jit, static_argnames=("n_rows",))
def _embed_concat(word_table_pad, label_table, word_ids, label_ids, *, n_rows):
    rows_per_w = n_rows // NW
    n_chunks = rows_per_w // CHUNK
    mesh = plsc.VectorSubcoreMesh(
        core_axis_name="c", subcore_axis_name="s", num_cores=NC, num_subcores=NS
    )

    @functools.partial(
        pl.kernel,
        out_type=jax.ShapeDtypeStruct((n_rows, PACK_DIM), jnp.float32),
        mesh=mesh,
        compiler_params=pltpu.CompilerParams(
            use_tc_tiling_on_sc=False, needs_layout_passes=False
        ),
        scratch_types=[
            pltpu.VMEM((CHUNK,), jnp.int32),
            pltpu.VMEM((CHUNK,), jnp.int32),
            pltpu.VMEM((CHUNK, WORD_PAD), jnp.float32),
            pltpu.VMEM((CHUNK, LABEL_DIM), jnp.float32),
            pltpu.SemaphoreType.DMA,
            pltpu.SemaphoreType.DMA,
        ],
    )
    def body(wtab, ltab, wids, lids, out, widx, lidx, wrows, lrows, sem0, sem1):
        wid = lax.axis_index("s") * NC + lax.axis_index("c")
        base0 = wid * rows_per_w

        def step(i, carry):
            base = base0 + i * CHUNK
            pltpu.sync_copy(wids.at[pl.ds(base, CHUNK)], widx)
            pltpu.sync_copy(lids.at[pl.ds(base, CHUNK)], lidx)
            cw = pltpu.async_copy(wtab.at[widx], wrows, sem0)
            cl = pltpu.async_copy(ltab.at[lidx], lrows, sem1)
            cw.wait()
            cl.wait()
            pltpu.sync_copy(wrows, out.at[pl.ds(base, CHUNK), pl.ds(0, WORD_PAD)])
            pltpu.sync_copy(lrows, out.at[pl.ds(base, CHUNK), pl.ds(WORD_PAD, LABEL_DIM)])
            return carry

        lax.fori_loop(0, n_chunks, step, 0)

    return body(word_table_pad, label_table, word_ids, label_ids)


_PAD_BLOCK = 4096


def _pad_body(x_ref, o_ref):
    o_ref[...] = jnp.concatenate(
        [x_ref[...], jnp.zeros((x_ref.shape[0], WORD_PAD - WORD_DIM), jnp.float32)],
        axis=1,
    )


def _pad_tc(word_table):
    v = word_table.shape[0]
    grid = (v + _PAD_BLOCK - 1) // _PAD_BLOCK
    return pl.pallas_call(
        _pad_body,
        grid=(grid,),
        in_specs=[pl.BlockSpec((_PAD_BLOCK, WORD_DIM), lambda i: (i, 0))],
        out_specs=pl.BlockSpec((_PAD_BLOCK, WORD_PAD), lambda i: (i, 0)),
        out_shape=jax.ShapeDtypeStruct((v, WORD_PAD), jnp.float32),
    )(word_table)


_STRIP_BLOCK = 2048


def _strip_body(x_ref, o_ref):
    x = x_ref[...]
    o_ref[...] = jnp.concatenate([x[:, :WORD_DIM], x[:, WORD_PAD:]], axis=1)


def _strip_tc(packed):
    n = packed.shape[0]
    grid = n // _STRIP_BLOCK
    return pl.pallas_call(
        _strip_body,
        grid=(grid,),
        in_specs=[pl.BlockSpec((_STRIP_BLOCK, PACK_DIM), lambda i: (i, 0))],
        out_specs=pl.BlockSpec((_STRIP_BLOCK, WORD_DIM + LABEL_DIM), lambda i: (i, 0)),
        out_shape=jax.ShapeDtypeStruct((n, WORD_DIM + LABEL_DIM), jnp.float32),
    )(packed)


def kernel(inputs, word_table, label_table):
    b, l, _ = inputs.shape
    n = b * l
    ids = inputs.reshape(n, 2)
    word_ids = ids[:, 0]
    label_ids = ids[:, 1]
    wt_pad = _pad_tc(word_table)
    h = n // 2
    packed_a = _embed_concat(
        wt_pad, label_table, word_ids[:h], label_ids[:h], n_rows=h
    )
    packed_b = _embed_concat(
        wt_pad, label_table, word_ids[h:], label_ids[h:], n_rows=h
    )
    out_a = _strip_tc(packed_a)
    out_b = _strip_tc(packed_b)
    out = jnp.concatenate([out_a, out_b], axis=0)
    return out.reshape(b, l, WORD_DIM + LABEL_DIM)


# speedup vs baseline: 1.2418x; 1.1218x over previous
import functools

import jax
import jax.numpy as jnp
from jax import lax
from jax.experimental import pallas as pl
from jax.experimental.pallas import tpu as pltpu
from jax.experimental.pallas import tpu_sc as plsc

NC = 2
NS = 16
NW = NC * NS

WORD_DIM = 100
WORD_PAD = 104
LABEL_DIM = 32
PACK_DIM = WORD_PAD + LABEL_DIM

CHUNK = 128


@functools.partial(jax.jit, static_argnames=("n_rows",))
def _embed_concat(word_table_pad, label_table, word_ids, label_ids, *, n_rows):
    rows_per_w = n_rows // NW
    n_chunks = rows_per_w // CHUNK
    mesh = plsc.VectorSubcoreMesh(
        core_axis_name="c", subcore_axis_name="s", num_cores=NC, num_subcores=NS
    )

    @functools.partial(
        pl.kernel,
        out_type=jax.ShapeDtypeStruct((n_rows, PACK_DIM), jnp.float32),
        mesh=mesh,
        compiler_params=pltpu.CompilerParams(
            use_tc_tiling_on_sc=False, needs_layout_passes=False
        ),
        scratch_types=[
            pltpu.VMEM((CHUNK,), jnp.int32),
            pltpu.VMEM((CHUNK,), jnp.int32),
            pltpu.VMEM((CHUNK, WORD_PAD), jnp.float32),
            pltpu.VMEM((CHUNK, LABEL_DIM), jnp.float32),
            pltpu.VMEM((CHUNK,), jnp.int32),
            pltpu.VMEM((CHUNK,), jnp.int32),
            pltpu.VMEM((CHUNK, WORD_PAD), jnp.float32),
            pltpu.VMEM((CHUNK, LABEL_DIM), jnp.float32),
            pltpu.SemaphoreType.DMA,
            pltpu.SemaphoreType.DMA,
            pltpu.SemaphoreType.DMA,
            pltpu.SemaphoreType.DMA,
        ],
    )
    def body(
        wtab, ltab, wids, lids, out,
        widx0, lidx0, wrows0, lrows0,
        widx1, lidx1, wrows1, lrows1,
        s0, s1, s2, s3,
    ):
        wid = lax.axis_index("s") * NC + lax.axis_index("c")
        base0 = wid * rows_per_w
        bufs = ((widx0, lidx0, wrows0, lrows0, s0, s1),
                (widx1, lidx1, wrows1, lrows1, s2, s3))

        def stage(k, bf):
            widx, lidx = bf[0], bf[1]
            base = base0 + k * CHUNK
            pltpu.sync_copy(wids.at[pl.ds(base, CHUNK)], widx)
            pltpu.sync_copy(lids.at[pl.ds(base, CHUNK)], lidx)

        def fire(bf):
            widx, lidx, wrows, lrows, sa, sb = bf
            pltpu.async_copy(wtab.at[widx], wrows, sa)
            pltpu.async_copy(ltab.at[lidx], lrows, sb)

        def wait(bf):
            widx, lidx, wrows, lrows, sa, sb = bf
            pltpu.make_async_copy(wtab.at[widx], wrows, sa).wait()
            pltpu.make_async_copy(ltab.at[lidx], lrows, sb).wait()

        def writeback(k, bf):
            wrows, lrows = bf[2], bf[3]
            base = base0 + k * CHUNK
            pltpu.sync_copy(wrows, out.at[pl.ds(base, CHUNK), pl.ds(0, WORD_PAD)])
            pltpu.sync_copy(
                lrows, out.at[pl.ds(base, CHUNK), pl.ds(WORD_PAD, LABEL_DIM)]
            )

        n_pairs = n_chunks // 2
        stage(0, bufs[0])
        fire(bufs[0])

        def step(j, carry):
            k0 = 2 * j
            stage(k0 + 1, bufs[1])
            fire(bufs[1])
            wait(bufs[0])
            writeback(k0, bufs[0])

            @pl.when(j < n_pairs - 1)
            def _():
                stage(k0 + 2, bufs[0])
                fire(bufs[0])

            wait(bufs[1])
            writeback(k0 + 1, bufs[1])
            return carry

        lax.fori_loop(0, n_pairs, step, 0)

    return body(word_table_pad, label_table, word_ids, label_ids)


_PAD_BLOCK = 4096


def _pad_body(x_ref, o_ref):
    o_ref[...] = jnp.concatenate(
        [x_ref[...], jnp.zeros((x_ref.shape[0], WORD_PAD - WORD_DIM), jnp.float32)],
        axis=1,
    )


def _pad_tc(word_table):
    v = word_table.shape[0]
    grid = (v + _PAD_BLOCK - 1) // _PAD_BLOCK
    return pl.pallas_call(
        _pad_body,
        grid=(grid,),
        in_specs=[pl.BlockSpec((_PAD_BLOCK, WORD_DIM), lambda i: (i, 0))],
        out_specs=pl.BlockSpec((_PAD_BLOCK, WORD_PAD), lambda i: (i, 0)),
        out_shape=jax.ShapeDtypeStruct((v, WORD_PAD), jnp.float32),
    )(word_table)


_STRIP_BLOCK = 2048


def _strip_body(x_ref, o_ref):
    x = x_ref[...]
    o_ref[...] = jnp.concatenate([x[:, :WORD_DIM], x[:, WORD_PAD:]], axis=1)


def _strip_tc(packed):
    n = packed.shape[0]
    grid = n // _STRIP_BLOCK
    return pl.pallas_call(
        _strip_body,
        grid=(grid,),
        in_specs=[pl.BlockSpec((_STRIP_BLOCK, PACK_DIM), lambda i: (i, 0))],
        out_specs=pl.BlockSpec((_STRIP_BLOCK, WORD_DIM + LABEL_DIM), lambda i: (i, 0)),
        out_shape=jax.ShapeDtypeStruct((n, WORD_DIM + LABEL_DIM), jnp.float32),
    )(packed)


def kernel(inputs, word_table, label_table):
    b, l, _ = inputs.shape
    ids = inputs.reshape(b * l, 2)
    word_ids = ids[:, 0]
    label_ids = ids[:, 1]
    wt_pad = _pad_tc(word_table)
    packed = _embed_concat(wt_pad, label_table, word_ids, label_ids, n_rows=b * l)
    out = _strip_tc(packed)
    return out.reshape(b, l, WORD_DIM + LABEL_DIM)
